# baseline (speedup 1.0000x reference)
import functools

import jax
import jax.numpy as jnp
from jax import lax
from jax.experimental import pallas as pl
from jax.experimental.pallas import tpu as pltpu
from jax.experimental.pallas import tpu_sc as plsc

_B, _S, _D = 2, 2048, 768
_E = 64
_DFF = 768
_NC = 10
_C = 80
_T = _B * _S
_TB = 512
_NBLK = _T // _TB
_TRASH = _E * _C
_NW = 32
_TPW = _T // _NW
_WCOLS = 128


def _router_body(x_ref, wg_ref, slot_ref, wbrow_ref, counts_ref, cnt_scr):
    i = pl.program_id(0)

    @pl.when(i == 0)
    def _():
        cnt_scr[...] = jnp.zeros_like(cnt_scr)

    xb = x_ref[...]
    logits = jnp.dot(xb, wg_ref[...], preferred_element_type=jnp.float32)
    lmax = jnp.max(logits, axis=1, keepdims=True)
    ex = jnp.exp(logits - lmax)
    probs = ex / jnp.sum(ex, axis=1, keepdims=True)
    gate = jnp.max(probs, axis=1, keepdims=True)
    lane = lax.broadcasted_iota(jnp.int32, (_TB, _E), 1).astype(jnp.float32)
    idx_f = jnp.min(jnp.where(probs == gate, lane, float(_E)),
                    axis=1, keepdims=True)
    oh = (lane == idx_f).astype(jnp.float32)

    r = lax.broadcasted_iota(jnp.int32, (_TB, _TB), 0)
    c = lax.broadcasted_iota(jnp.int32, (_TB, _TB), 1)
    ltri = (c < r).astype(jnp.float32)
    prefix = jnp.dot(ltri, oh, preferred_element_type=jnp.float32)

    cnt = cnt_scr[...]
    pos = jnp.sum(oh * (prefix + cnt), axis=1, keepdims=True)
    slot_f = jnp.where(pos < float(_C), idx_f * float(_C) + pos,
                       float(_TRASH))
    slot_ref[...] = slot_f.astype(jnp.int32)

    b_idx = (i * _TB >= _S).astype(jnp.int32)
    col = lax.broadcasted_iota(jnp.int32, (_TB, _WCOLS), 1)
    wbrow_ref[...] = jnp.where(col == b_idx, gate, 0.0)

    new_cnt = cnt + jnp.sum(oh, axis=0, keepdims=True)
    cnt_scr[...] = new_cnt
    counts_ref[...] = new_cnt.astype(jnp.int32)


_router = pl.pallas_call(
    _router_body,
    grid=(_NBLK,),
    in_specs=[
        pl.BlockSpec((_TB, _D), lambda i: (i, 0)),
        pl.BlockSpec((_D, _E), lambda i: (0, 0)),
    ],
    out_specs=[
        pl.BlockSpec((_TB, 1), lambda i: (i, 0)),
        pl.BlockSpec((_TB, _WCOLS), lambda i: (i, 0)),
        pl.BlockSpec((1, _E), lambda i: (0, 0)),
    ],
    out_shape=[
        jax.ShapeDtypeStruct((_T, 1), jnp.int32),
        jax.ShapeDtypeStruct((_T, _WCOLS), jnp.float32),
        jax.ShapeDtypeStruct((1, _E), jnp.int32),
    ],
    scratch_shapes=[pltpu.VMEM((1, _E), jnp.float32)],
)


_CH = 4
_CT = _TPW // _CH


def _dispatch_body(t_hbm, wbrow_hbm, slot_hbm, disp_hbm, wb_hbm,
                   idx_v, rows_v, wbr_v, lsem0, lsem1, ssem, wsem, isem):
    wid = lax.axis_index("s") * 2 + lax.axis_index("c")
    base = wid * _TPW
    ldis = [pltpu.async_copy(slot_hbm.at[pl.ds(base + k * _CT, _CT)],
                             idx_v.at[k], isem) for k in range(_CH)]
    ldws = [pltpu.async_copy(wbrow_hbm.at[pl.ds(base + k * _CT, _CT)],
                             wbr_v.at[k], wsem) for k in range(_CH)]
    lsems = (lsem0, lsem1)
    lds = [None] * _CH
    for k in range(2):
        lds[k] = pltpu.async_copy(t_hbm.at[pl.ds(base + k * _CT, _CT)],
                                  rows_v.at[k], lsems[k % 2])
    for ld in ldis:
        ld.wait()
    scats = []
    for k in range(_CH):
        lds[k].wait()
        if k + 2 < _CH:
            lds[k + 2] = pltpu.async_copy(
                t_hbm.at[pl.ds(base + (k + 2) * _CT, _CT)],
                rows_v.at[k + 2], lsems[k % 2])
        scats.append(pltpu.async_copy(rows_v.at[k], disp_hbm.at[idx_v.at[k]],
                                      ssem))
    for ld in ldws:
        ld.wait()
    cpws = [pltpu.async_copy(wbr_v.at[k], wb_hbm.at[idx_v.at[k]], wsem)
            for k in range(_CH)]
    for cp in scats:
        cp.wait()
    for cp in cpws:
        cp.wait()


@functools.cache
def _make_dispatch():
    return pl.kernel(
        _dispatch_body,
        out_type=[
            jax.ShapeDtypeStruct((_TRASH + 1, _D), jnp.float32),
            jax.ShapeDtypeStruct((_TRASH + 1, _WCOLS), jnp.float32),
        ],
        mesh=plsc.VectorSubcoreMesh(core_axis_name="c",
                                    subcore_axis_name="s"),
        scratch_types=[
            pltpu.VMEM((_CH, _CT), jnp.int32),
            pltpu.VMEM((_CH, _CT, _D), jnp.float32),
            pltpu.VMEM((_CH, _CT, _WCOLS), jnp.float32),
            pltpu.SemaphoreType.DMA,
            pltpu.SemaphoreType.DMA,
            pltpu.SemaphoreType.DMA,
            pltpu.SemaphoreType.DMA,
            pltpu.SemaphoreType.DMA,
        ],
    )


def _ffn_body(cnt_smem, disp_ref, wb_ref, w1_ref, b1_ref, w2_ref, b2_ref,
              wc_ref, bc_ref, out_ref, acc):
    e = pl.program_id(0)
    nfill = jnp.minimum(cnt_smem[e], _C)
    rmask_d = lax.broadcasted_iota(jnp.int32, (_C, _D), 0) < nfill
    dispc = jnp.where(rmask_d, disp_ref[...], 0.0)
    rmask_w = lax.broadcasted_iota(jnp.int32, (_C, _WCOLS), 0) < nfill
    wbc = jnp.where(rmask_w, wb_ref[...], 0.0)

    h = jnp.maximum(
        jnp.dot(dispc, w1_ref[0], preferred_element_type=jnp.float32)
        + b1_ref[0], 0.0)
    hb = lax.dot_general(wbc, h, (((0,), (0,)), ((), ())),
                         preferred_element_type=jnp.float32)
    contrib = jnp.dot(hb, w2_ref[0], preferred_element_type=jnp.float32)
    ones_col = jnp.ones((_C, 1), jnp.float32)
    s_col = lax.dot_general(wbc, ones_col, (((0,), (0,)), ((), ())),
                            preferred_element_type=jnp.float32)
    contrib = contrib + s_col * b2_ref[0]

    @pl.when(e == 0)
    def _():
        acc[...] = contrib

    @pl.when(e != 0)
    def _():
        acc[...] = acc[...] + contrib

    @pl.when(e == _E - 1)
    def _():
        pooled = acc[...] * (1.0 / _S)
        out_ref[...] = (jnp.dot(pooled, wc_ref[...],
                                preferred_element_type=jnp.float32)
                        + bc_ref[...])


_ffn = pl.pallas_call(
    _ffn_body,
    grid_spec=pltpu.PrefetchScalarGridSpec(
        num_scalar_prefetch=1,
        grid=(_E,),
        in_specs=[
            pl.BlockSpec((_C, _D), lambda e, cnt: (e, 0)),
            pl.BlockSpec((_C, _WCOLS), lambda e, cnt: (e, 0)),
            pl.BlockSpec((1, _D, _DFF), lambda e, cnt: (e, 0, 0)),
            pl.BlockSpec((1, 1, _DFF), lambda e, cnt: (e, 0, 0)),
            pl.BlockSpec((1, _DFF, _D), lambda e, cnt: (e, 0, 0)),
            pl.BlockSpec((1, 1, _D), lambda e, cnt: (e, 0, 0)),
            pl.BlockSpec((_D, _NC), lambda e, cnt: (0, 0)),
            pl.BlockSpec((1, _NC), lambda e, cnt: (0, 0)),
        ],
        out_specs=pl.BlockSpec((_WCOLS, _NC), lambda e, cnt: (0, 0)),
        scratch_shapes=[pltpu.VMEM((_WCOLS, _D), jnp.float32)],
    ),
    out_shape=jax.ShapeDtypeStruct((_WCOLS, _NC), jnp.float32),
)


def kernel(x, Wg, W1, b1, W2, b2, Wc, bc):
    t = x.reshape(_T, _D)
    slot2, wbrow, counts = _router(t, Wg)
    slot = slot2.reshape(_T)
    cnt_i = counts.reshape(_E)
    disp, wb = _make_dispatch()(t, wbrow, slot)
    out16 = _ffn(cnt_i, disp, wb, W1, b1.reshape(_E, 1, _DFF), W2,
                 b2.reshape(_E, 1, _D), Wc, bc.reshape(1, _NC))
    return out16[0:_B, :]


# speedup vs baseline: 1.1223x; 1.1223x over previous
import functools

import jax
import jax.numpy as jnp
from jax import lax
from jax.experimental import pallas as pl
from jax.experimental.pallas import tpu as pltpu
from jax.experimental.pallas import tpu_sc as plsc

_B, _S, _D = 2, 2048, 768
_E = 64
_DFF = 768
_NC = 10
_C = 80
_T = _B * _S
_TB = 512
_NBLK = _T // _TB
_TRASH = _E * _C
_NW = 32
_TPW = _T // _NW
_WCOLS = 128


def _router_body(x_ref, wg_ref, slot_ref, wbrow_ref, counts_ref, cnt_scr):
    i = pl.program_id(0)

    @pl.when(i == 0)
    def _():
        cnt_scr[...] = jnp.zeros_like(cnt_scr)

    xb = x_ref[...]
    logits = jnp.dot(xb, wg_ref[...], preferred_element_type=jnp.float32)
    lmax = jnp.max(logits, axis=1, keepdims=True)
    ex = jnp.exp(logits - lmax)
    exmax = jnp.max(ex, axis=1, keepdims=True)
    gate = exmax / jnp.sum(ex, axis=1, keepdims=True)
    lane = lax.broadcasted_iota(jnp.int32, (_TB, _E), 1).astype(jnp.float32)
    idx_f = jnp.min(jnp.where(ex == exmax, lane, float(_E)),
                    axis=1, keepdims=True)
    oh = (lane == idx_f).astype(jnp.float32)

    r = lax.broadcasted_iota(jnp.int32, (_TB, _TB), 0)
    c = lax.broadcasted_iota(jnp.int32, (_TB, _TB), 1)
    ltri = (c < r).astype(jnp.float32)
    prefix = jnp.dot(ltri, oh, preferred_element_type=jnp.float32)

    cnt = cnt_scr[...]
    pos = jnp.sum(oh * (prefix + cnt), axis=1, keepdims=True)
    slot_f = jnp.where(pos < float(_C), idx_f * float(_C) + pos,
                       float(_TRASH))
    slot_ref[pl.ds(i * (_TB // 128), _TB // 128), :] = (
        slot_f.astype(jnp.int32).reshape(_TB // 128, 128))

    b_idx = (i * _TB >= _S).astype(jnp.int32)
    col = lax.broadcasted_iota(jnp.int32, (_TB, _WCOLS), 1)
    wbrow_ref[...] = jnp.where(col == b_idx, gate, 0.0)

    new_cnt = cnt + jnp.sum(oh, axis=0, keepdims=True)
    cnt_scr[...] = new_cnt
    counts_ref[...] = new_cnt.astype(jnp.int32)


_router = pl.pallas_call(
    _router_body,
    grid=(_NBLK,),
    in_specs=[
        pl.BlockSpec((_TB, _D), lambda i: (i, 0)),
        pl.BlockSpec((_D, _E), lambda i: (0, 0)),
    ],
    out_specs=[
        pl.BlockSpec((_T // 128, 128), lambda i: (0, 0)),
        pl.BlockSpec((_TB, _WCOLS), lambda i: (i, 0)),
        pl.BlockSpec((1, _E), lambda i: (0, 0)),
    ],
    out_shape=[
        jax.ShapeDtypeStruct((_T // 128, 128), jnp.int32),
        jax.ShapeDtypeStruct((_T, _WCOLS), jnp.float32),
        jax.ShapeDtypeStruct((1, _E), jnp.int32),
    ],
    scratch_shapes=[pltpu.VMEM((1, _E), jnp.float32)],
)


def _dispatch_body(t_hbm, wbrow_hbm, slot_hbm, disp_hbm, wb_hbm,
                   idx_v, rows_v, wbr_v, sem1, sem2, sem3):
    wid = lax.axis_index("s") * 2 + lax.axis_index("c")
    base = wid * _TPW
    ld1 = pltpu.async_copy(slot_hbm.at[pl.ds(base, _TPW)], idx_v, sem1)
    ld2 = pltpu.async_copy(t_hbm.at[pl.ds(base, _TPW)], rows_v, sem2)
    ld3 = pltpu.async_copy(wbrow_hbm.at[pl.ds(base, _TPW)], wbr_v, sem3)
    ld1.wait()
    ld2.wait()
    ld3.wait()
    cp1 = pltpu.async_copy(rows_v, disp_hbm.at[idx_v], sem1)
    cp2 = pltpu.async_copy(wbr_v, wb_hbm.at[idx_v], sem2)
    cp1.wait()
    cp2.wait()


@functools.cache
def _make_dispatch():
    return pl.kernel(
        _dispatch_body,
        out_type=[
            jax.ShapeDtypeStruct((_TRASH + 1, _D), jnp.float32),
            jax.ShapeDtypeStruct((_TRASH + 1, _WCOLS), jnp.float32),
        ],
        mesh=plsc.VectorSubcoreMesh(core_axis_name="c",
                                    subcore_axis_name="s"),
        scratch_types=[
            pltpu.VMEM((_TPW,), jnp.int32),
            pltpu.VMEM((_TPW, _D), jnp.float32),
            pltpu.VMEM((_TPW, _WCOLS), jnp.float32),
            pltpu.SemaphoreType.DMA,
            pltpu.SemaphoreType.DMA,
            pltpu.SemaphoreType.DMA,
        ],
    )


_EB = 4


def _ffn_body(cnt_smem, disp_ref, wb_ref, w1_ref, b1_ref, w2_ref, b2_ref,
              wc_ref, bc_ref, out_ref, acc):
    i = pl.program_id(0)
    contrib = None
    for j in range(_EB):
        nfill = jnp.minimum(cnt_smem[i * _EB + j], _C)
        rmask_d = lax.broadcasted_iota(jnp.int32, (_C, _D), 0) < nfill
        dispc = jnp.where(rmask_d, disp_ref[pl.ds(j * _C, _C), :], 0.0)
        rmask_w = lax.broadcasted_iota(jnp.int32, (_C, _WCOLS), 0) < nfill
        wbc = jnp.where(rmask_w, wb_ref[pl.ds(j * _C, _C), :], 0.0)

        h = jnp.maximum(
            jnp.dot(dispc, w1_ref[j], preferred_element_type=jnp.float32)
            + b1_ref[j], 0.0)
        hb = lax.dot_general(wbc, h, (((0,), (0,)), ((), ())),
                             preferred_element_type=jnp.float32)
        cj = jnp.dot(hb, w2_ref[j], preferred_element_type=jnp.float32)
        ones_col = jnp.ones((_C, 1), jnp.float32)
        s_col = lax.dot_general(wbc, ones_col, (((0,), (0,)), ((), ())),
                                preferred_element_type=jnp.float32)
        cj = cj + s_col * b2_ref[j]
        contrib = cj if contrib is None else contrib + cj

    @pl.when(i == 0)
    def _():
        acc[...] = contrib

    @pl.when(i != 0)
    def _():
        acc[...] = acc[...] + contrib

    @pl.when(i == _E // _EB - 1)
    def _():
        pooled = acc[...] * (1.0 / _S)
        out_ref[...] = (jnp.dot(pooled, wc_ref[...],
                                preferred_element_type=jnp.float32)
                        + bc_ref[...])


_ffn = pl.pallas_call(
    _ffn_body,
    grid_spec=pltpu.PrefetchScalarGridSpec(
        num_scalar_prefetch=1,
        grid=(_E // _EB,),
        in_specs=[
            pl.BlockSpec((_EB * _C, _D), lambda e, cnt: (e, 0)),
            pl.BlockSpec((_EB * _C, _WCOLS), lambda e, cnt: (e, 0)),
            pl.BlockSpec((_EB, _D, _DFF), lambda e, cnt: (e, 0, 0)),
            pl.BlockSpec((_EB, 1, _DFF), lambda e, cnt: (e, 0, 0)),
            pl.BlockSpec((_EB, _DFF, _D), lambda e, cnt: (e, 0, 0)),
            pl.BlockSpec((_EB, 1, _D), lambda e, cnt: (e, 0, 0)),
            pl.BlockSpec((_D, _NC), lambda e, cnt: (0, 0)),
            pl.BlockSpec((1, _NC), lambda e, cnt: (0, 0)),
        ],
        out_specs=pl.BlockSpec((_WCOLS, _NC), lambda e, cnt: (0, 0)),
        scratch_shapes=[pltpu.VMEM((_WCOLS, _D), jnp.float32)],
    ),
    out_shape=jax.ShapeDtypeStruct((_WCOLS, _NC), jnp.float32),
)


def kernel(x, Wg, W1, b1, W2, b2, Wc, bc):
    t = x.reshape(_T, _D)
    slot2, wbrow, counts = _router(t, Wg)
    slot = slot2.reshape(_T)
    cnt_i = counts.reshape(_E)
    disp, wb = _make_dispatch()(t, wbrow, slot)
    out16 = _ffn(cnt_i, disp, wb, W1, b1.reshape(_E, 1, _DFF), W2,
                 b2.reshape(_E, 1, _D), Wc, bc.reshape(1, _NC))
    return out16[0:_B, :]


# speedup vs baseline: 1.1560x; 1.0300x over previous
import functools

import jax
import jax.numpy as jnp
from jax import lax
from jax.experimental import pallas as pl
from jax.experimental.pallas import tpu as pltpu
from jax.experimental.pallas import tpu_sc as plsc

_B, _S, _D = 2, 2048, 768
_E = 64
_DFF = 768
_NC = 10
_C = 80
_T = _B * _S
_TB = 512
_NBLK = _T // _TB
_TRASH = _E * _C
_NW = 32
_TPW = _T // _NW
_WCOLS = 128


def _router_body(x_ref, wg_ref, slot_ref, wbrow_ref, counts_ref, cnt_scr):
    i = pl.program_id(0)

    @pl.when(i == 0)
    def _():
        cnt_scr[...] = jnp.zeros_like(cnt_scr)

    xb = x_ref[...]
    logits = jnp.dot(xb, wg_ref[...], preferred_element_type=jnp.float32)
    lmax = jnp.max(logits, axis=1, keepdims=True)
    ex = jnp.exp(logits - lmax)
    exmax = jnp.max(ex, axis=1, keepdims=True)
    gate = exmax / jnp.sum(ex, axis=1, keepdims=True)
    lane = lax.broadcasted_iota(jnp.int32, (_TB, _E), 1).astype(jnp.float32)
    idx_f = jnp.min(jnp.where(ex == exmax, lane, float(_E)),
                    axis=1, keepdims=True)
    oh = (lane == idx_f).astype(jnp.float32)

    r = lax.broadcasted_iota(jnp.int32, (_TB, _TB), 0)
    c = lax.broadcasted_iota(jnp.int32, (_TB, _TB), 1)
    ltri = (c < r).astype(jnp.float32)
    prefix = jnp.dot(ltri, oh, preferred_element_type=jnp.float32)

    cnt = cnt_scr[...]
    pos = jnp.sum(oh * (prefix + cnt), axis=1, keepdims=True)
    slot_f = jnp.where(pos < float(_C), idx_f * float(_C) + pos,
                       float(_TRASH))
    slot_ref[pl.ds(i * (_TB // 128), _TB // 128), :] = (
        slot_f.astype(jnp.int32).reshape(_TB // 128, 128))

    b_idx = (i * _TB >= _S).astype(jnp.int32)
    col = lax.broadcasted_iota(jnp.int32, (_TB, _WCOLS), 1)
    wbrow_ref[...] = jnp.where(col == b_idx, gate, 0.0)

    new_cnt = cnt + jnp.sum(oh, axis=0, keepdims=True)
    cnt_scr[...] = new_cnt
    counts_ref[...] = new_cnt.astype(jnp.int32)


_router = pl.pallas_call(
    _router_body,
    grid=(_NBLK,),
    in_specs=[
        pl.BlockSpec((_TB, _D), lambda i: (i, 0)),
        pl.BlockSpec((_D, _E), lambda i: (0, 0)),
    ],
    out_specs=[
        pl.BlockSpec((_T // 128, 128), lambda i: (0, 0)),
        pl.BlockSpec((_TB, _WCOLS), lambda i: (i, 0)),
        pl.BlockSpec((1, _E), lambda i: (0, 0)),
    ],
    out_shape=[
        jax.ShapeDtypeStruct((_T // 128, 128), jnp.int32),
        jax.ShapeDtypeStruct((_T, _WCOLS), jnp.float32),
        jax.ShapeDtypeStruct((1, _E), jnp.int32),
    ],
    scratch_shapes=[pltpu.VMEM((1, _E), jnp.float32)],
)


def _dispatch_body(t_hbm, wbrow_hbm, slot_hbm, disp_hbm, wb_hbm,
                   idx_v, rows_v, wbr_v, sem1, sem2, sem3):
    wid = lax.axis_index("s") * 2 + lax.axis_index("c")
    base = wid * _TPW
    ld1 = pltpu.async_copy(slot_hbm.at[pl.ds(base, _TPW)], idx_v, sem1)
    ld2 = pltpu.async_copy(t_hbm.at[pl.ds(base, _TPW)], rows_v, sem2)
    ld3 = pltpu.async_copy(wbrow_hbm.at[pl.ds(base, _TPW)], wbr_v, sem3)
    ld1.wait()
    ld2.wait()
    ld3.wait()
    cp1 = pltpu.async_copy(rows_v, disp_hbm.at[idx_v], sem1)
    cp2 = pltpu.async_copy(wbr_v, wb_hbm.at[idx_v], sem2)
    cp1.wait()
    cp2.wait()


@functools.cache
def _make_dispatch():
    return pl.kernel(
        _dispatch_body,
        out_type=[
            jax.ShapeDtypeStruct((_TRASH + 1, _D), jnp.float32),
            jax.ShapeDtypeStruct((_TRASH + 1, _WCOLS), jnp.float32),
        ],
        mesh=plsc.VectorSubcoreMesh(core_axis_name="c",
                                    subcore_axis_name="s"),
        scratch_types=[
            pltpu.VMEM((_TPW,), jnp.int32),
            pltpu.VMEM((_TPW, _D), jnp.float32),
            pltpu.VMEM((_TPW, _WCOLS), jnp.float32),
            pltpu.SemaphoreType.DMA,
            pltpu.SemaphoreType.DMA,
            pltpu.SemaphoreType.DMA,
        ],
    )


_EB = 2


def _ffn_body(cnt_smem, disp_ref, wb_ref, w1_ref, b1_ref, w2_ref, b2_ref,
              wc_ref, bc_ref, out_ref, acc):
    i = pl.program_id(0)
    contrib = None
    for j in range(_EB):
        nfill = jnp.minimum(cnt_smem[i * _EB + j], _C)
        rmask_d = lax.broadcasted_iota(jnp.int32, (_C, _D), 0) < nfill
        dispc = jnp.where(rmask_d, disp_ref[pl.ds(j * _C, _C), :], 0.0)
        rmask_w = lax.broadcasted_iota(jnp.int32, (_C, _WCOLS), 0) < nfill
        wbc = jnp.where(rmask_w, wb_ref[pl.ds(j * _C, _C), :], 0.0)

        h = jnp.maximum(
            jnp.dot(dispc, w1_ref[j], preferred_element_type=jnp.float32)
            + b1_ref[j], 0.0)
        hb = lax.dot_general(wbc, h, (((0,), (0,)), ((), ())),
                             preferred_element_type=jnp.float32)
        cj = jnp.dot(hb, w2_ref[j], preferred_element_type=jnp.float32)
        ones_col = jnp.ones((_C, 1), jnp.float32)
        s_col = lax.dot_general(wbc, ones_col, (((0,), (0,)), ((), ())),
                                preferred_element_type=jnp.float32)
        cj = cj + s_col * b2_ref[j]
        contrib = cj if contrib is None else contrib + cj

    @pl.when(i == 0)
    def _():
        acc[...] = contrib

    @pl.when(i != 0)
    def _():
        acc[...] = acc[...] + contrib

    @pl.when(i == _E // _EB - 1)
    def _():
        pooled = acc[...] * (1.0 / _S)
        out_ref[...] = (jnp.dot(pooled, wc_ref[...],
                                preferred_element_type=jnp.float32)
                        + bc_ref[...])


_ffn = pl.pallas_call(
    _ffn_body,
    grid_spec=pltpu.PrefetchScalarGridSpec(
        num_scalar_prefetch=1,
        grid=(_E // _EB,),
        in_specs=[
            pl.BlockSpec((_EB * _C, _D), lambda e, cnt: (e, 0)),
            pl.BlockSpec((_EB * _C, _WCOLS), lambda e, cnt: (e, 0)),
            pl.BlockSpec((_EB, _D, _DFF), lambda e, cnt: (e, 0, 0)),
            pl.BlockSpec((_EB, 1, _DFF), lambda e, cnt: (e, 0, 0)),
            pl.BlockSpec((_EB, _DFF, _D), lambda e, cnt: (e, 0, 0)),
            pl.BlockSpec((_EB, 1, _D), lambda e, cnt: (e, 0, 0)),
            pl.BlockSpec((_D, _NC), lambda e, cnt: (0, 0)),
            pl.BlockSpec((1, _NC), lambda e, cnt: (0, 0)),
        ],
        out_specs=pl.BlockSpec((_WCOLS, _NC), lambda e, cnt: (0, 0)),
        scratch_shapes=[pltpu.VMEM((_WCOLS, _D), jnp.float32)],
    ),
    out_shape=jax.ShapeDtypeStruct((_WCOLS, _NC), jnp.float32),
)


def kernel(x, Wg, W1, b1, W2, b2, Wc, bc):
    t = x.reshape(_T, _D)
    slot2, wbrow, counts = _router(t, Wg)
    slot = slot2.reshape(_T)
    cnt_i = counts.reshape(_E)
    disp, wb = _make_dispatch()(t, wbrow, slot)
    out16 = _ffn(cnt_i, disp, wb, W1, b1.reshape(_E, 1, _DFF), W2,
                 b2.reshape(_E, 1, _D), Wc, bc.reshape(1, _NC))
    return out16[0:_B, :]


# speedup vs baseline: 1.1560x; 1.0001x over previous
import functools

import jax
import jax.numpy as jnp
from jax import lax
from jax.experimental import pallas as pl
from jax.experimental.pallas import tpu as pltpu
from jax.experimental.pallas import tpu_sc as plsc

_B, _S, _D = 2, 2048, 768
_E = 64
_DFF = 768
_NC = 10
_C = 80
_T = _B * _S
_TB = 512
_NBLK = _T // _TB
_TRASH = _E * _C
_NW = 32
_TPW = _T // _NW
_WCOLS = 128


def _router_body(x_ref, wg_ref, slot_ref, wbrow_ref, counts_ref, cnt_scr):
    i = pl.program_id(0)

    @pl.when(i == 0)
    def _():
        cnt_scr[...] = jnp.zeros_like(cnt_scr)

    xb = x_ref[...]
    logits = jnp.dot(xb, wg_ref[...], preferred_element_type=jnp.float32)
    lmax = jnp.max(logits, axis=1, keepdims=True)
    ex = jnp.exp(logits - lmax)
    exmax = jnp.max(ex, axis=1, keepdims=True)
    gate = exmax / jnp.sum(ex, axis=1, keepdims=True)
    lane = lax.broadcasted_iota(jnp.int32, (_TB, _E), 1).astype(jnp.float32)
    idx_f = jnp.min(jnp.where(ex == exmax, lane, float(_E)),
                    axis=1, keepdims=True)
    oh = (lane == idx_f).astype(jnp.float32)

    r = lax.broadcasted_iota(jnp.int32, (_TB, _TB), 0)
    c = lax.broadcasted_iota(jnp.int32, (_TB, _TB), 1)
    ltri = (c < r).astype(jnp.float32)
    prefix = jnp.dot(ltri, oh, preferred_element_type=jnp.float32)

    cnt = cnt_scr[...]
    pos = jnp.sum(oh * (prefix + cnt), axis=1, keepdims=True)
    slot_f = jnp.where(pos < float(_C), idx_f * float(_C) + pos,
                       float(_TRASH))
    slot_ref[pl.ds(i * (_TB // 128), _TB // 128), :] = (
        slot_f.astype(jnp.int32).reshape(_TB // 128, 128))

    b_idx = (i * _TB >= _S).astype(jnp.int32)
    col = lax.broadcasted_iota(jnp.int32, (_TB, _WCOLS), 1)
    wbrow_ref[...] = jnp.where(col == b_idx, gate, 0.0)

    new_cnt = cnt + jnp.sum(oh, axis=0, keepdims=True)
    cnt_scr[...] = new_cnt
    counts_ref[...] = new_cnt.astype(jnp.int32)


_router = pl.pallas_call(
    _router_body,
    grid=(_NBLK,),
    in_specs=[
        pl.BlockSpec((_TB, _D), lambda i: (i, 0)),
        pl.BlockSpec((_D, _E), lambda i: (0, 0)),
    ],
    out_specs=[
        pl.BlockSpec((_T // 128, 128), lambda i: (0, 0)),
        pl.BlockSpec((_TB, _WCOLS), lambda i: (i, 0)),
        pl.BlockSpec((1, _E), lambda i: (0, 0)),
    ],
    out_shape=[
        jax.ShapeDtypeStruct((_T // 128, 128), jnp.int32),
        jax.ShapeDtypeStruct((_T, _WCOLS), jnp.float32),
        jax.ShapeDtypeStruct((1, _E), jnp.int32),
    ],
    scratch_shapes=[pltpu.VMEM((1, _E), jnp.float32)],
)


_HT = _TPW // 2


def _dispatch_body(t_hbm, wbrow_hbm, slot_hbm, disp_hbm, wb_hbm,
                   idx_v, rows_v, wbr_v, sem1, sem2, sem3, sem4):
    wid = lax.axis_index("s") * 2 + lax.axis_index("c")
    base = wid * _TPW
    ld1a = pltpu.async_copy(slot_hbm.at[pl.ds(base, _HT)], idx_v.at[0], sem1)
    ld1b = pltpu.async_copy(slot_hbm.at[pl.ds(base + _HT, _HT)],
                            idx_v.at[1], sem1)
    ldr0 = pltpu.async_copy(t_hbm.at[pl.ds(base, _HT)], rows_v.at[0], sem2)
    ldw0 = pltpu.async_copy(wbrow_hbm.at[pl.ds(base, _HT)], wbr_v.at[0], sem3)
    ldw1 = pltpu.async_copy(wbrow_hbm.at[pl.ds(base + _HT, _HT)],
                            wbr_v.at[1], sem3)
    ldr1 = pltpu.async_copy(t_hbm.at[pl.ds(base + _HT, _HT)], rows_v.at[1],
                            sem4)
    ld1a.wait()
    ld1b.wait()
    ldr0.wait()
    cp0 = pltpu.async_copy(rows_v.at[0], disp_hbm.at[idx_v.at[0]], sem2)
    ldw0.wait()
    ldw1.wait()
    cpw0 = pltpu.async_copy(wbr_v.at[0], wb_hbm.at[idx_v.at[0]], sem3)
    cpw1 = pltpu.async_copy(wbr_v.at[1], wb_hbm.at[idx_v.at[1]], sem3)
    ldr1.wait()
    cp1 = pltpu.async_copy(rows_v.at[1], disp_hbm.at[idx_v.at[1]], sem4)
    cp0.wait()
    cpw0.wait()
    cpw1.wait()
    cp1.wait()


@functools.cache
def _make_dispatch():
    return pl.kernel(
        _dispatch_body,
        out_type=[
            jax.ShapeDtypeStruct((_TRASH + 1, _D), jnp.float32),
            jax.ShapeDtypeStruct((_TRASH + 1, _WCOLS), jnp.float32),
        ],
        mesh=plsc.VectorSubcoreMesh(core_axis_name="c",
                                    subcore_axis_name="s"),
        scratch_types=[
            pltpu.VMEM((2, _HT), jnp.int32),
            pltpu.VMEM((2, _HT, _D), jnp.float32),
            pltpu.VMEM((2, _HT, _WCOLS), jnp.float32),
            pltpu.SemaphoreType.DMA,
            pltpu.SemaphoreType.DMA,
            pltpu.SemaphoreType.DMA,
            pltpu.SemaphoreType.DMA,
        ],
    )


_EB = 2


def _ffn_body(cnt_smem, disp_ref, wb_ref, w1_ref, b1_ref, w2_ref, b2_ref,
              wc_ref, bc_ref, out_ref, acc):
    i = pl.program_id(0)
    contrib = None
    for j in range(_EB):
        nfill = jnp.minimum(cnt_smem[i * _EB + j], _C)
        rmask_d = lax.broadcasted_iota(jnp.int32, (_C, _D), 0) < nfill
        dispc = jnp.where(rmask_d, disp_ref[pl.ds(j * _C, _C), :], 0.0)
        rmask_w = lax.broadcasted_iota(jnp.int32, (_C, _WCOLS), 0) < nfill
        wbc = jnp.where(rmask_w, wb_ref[pl.ds(j * _C, _C), :], 0.0)

        h = jnp.maximum(
            jnp.dot(dispc, w1_ref[j], preferred_element_type=jnp.float32)
            + b1_ref[j], 0.0)
        hb = lax.dot_general(wbc, h, (((0,), (0,)), ((), ())),
                             preferred_element_type=jnp.float32)
        cj = jnp.dot(hb, w2_ref[j], preferred_element_type=jnp.float32)
        ones_col = jnp.ones((_C, 1), jnp.float32)
        s_col = lax.dot_general(wbc, ones_col, (((0,), (0,)), ((), ())),
                                preferred_element_type=jnp.float32)
        cj = cj + s_col * b2_ref[j]
        contrib = cj if contrib is None else contrib + cj

    @pl.when(i == 0)
    def _():
        acc[...] = contrib

    @pl.when(i != 0)
    def _():
        acc[...] = acc[...] + contrib

    @pl.when(i == _E // _EB - 1)
    def _():
        pooled = acc[...] * (1.0 / _S)
        out_ref[...] = (jnp.dot(pooled, wc_ref[...],
                                preferred_element_type=jnp.float32)
                        + bc_ref[...])


_ffn = pl.pallas_call(
    _ffn_body,
    grid_spec=pltpu.PrefetchScalarGridSpec(
        num_scalar_prefetch=1,
        grid=(_E // _EB,),
        in_specs=[
            pl.BlockSpec((_EB * _C, _D), lambda e, cnt: (e, 0)),
            pl.BlockSpec((_EB * _C, _WCOLS), lambda e, cnt: (e, 0)),
            pl.BlockSpec((_EB, _D, _DFF), lambda e, cnt: (e, 0, 0)),
            pl.BlockSpec((_EB, 1, _DFF), lambda e, cnt: (e, 0, 0)),
            pl.BlockSpec((_EB, _DFF, _D), lambda e, cnt: (e, 0, 0)),
            pl.BlockSpec((_EB, 1, _D), lambda e, cnt: (e, 0, 0)),
            pl.BlockSpec((_D, _NC), lambda e, cnt: (0, 0)),
            pl.BlockSpec((1, _NC), lambda e, cnt: (0, 0)),
        ],
        out_specs=pl.BlockSpec((_WCOLS, _NC), lambda e, cnt: (0, 0)),
        scratch_shapes=[pltpu.VMEM((_WCOLS, _D), jnp.float32)],
    ),
    out_shape=jax.ShapeDtypeStruct((_WCOLS, _NC), jnp.float32),
)


def kernel(x, Wg, W1, b1, W2, b2, Wc, bc):
    t = x.reshape(_T, _D)
    slot2, wbrow, counts = _router(t, Wg)
    slot = slot2.reshape(_T)
    cnt_i = counts.reshape(_E)
    disp, wb = _make_dispatch()(t, wbrow, slot)
    out16 = _ffn(cnt_i, disp, wb, W1, b1.reshape(_E, 1, _DFF), W2,
                 b2.reshape(_E, 1, _D), Wc, bc.reshape(1, _NC))
    return out16[0:_B, :]


# speedup vs baseline: 1.1677x; 1.0101x over previous
import functools

import jax
import jax.numpy as jnp
from jax import lax
from jax.experimental import pallas as pl
from jax.experimental.pallas import tpu as pltpu
from jax.experimental.pallas import tpu_sc as plsc

_B, _S, _D = 2, 2048, 768
_E = 64
_DFF = 768
_NC = 10
_C = 80
_T = _B * _S
_TB = 512
_NBLK = _T // _TB
_TRASH = _E * _C
_NW = 32
_TPW = _T // _NW
_WCOLS = 128


def _router_body(x_ref, wg_ref, slot_ref, wbrow_ref, counts_ref, cnt_scr):
    i = pl.program_id(0)

    @pl.when(i == 0)
    def _():
        cnt_scr[...] = jnp.zeros_like(cnt_scr)

    xb = x_ref[...]
    logits = jnp.dot(xb, wg_ref[...], preferred_element_type=jnp.float32)
    lmax = jnp.max(logits, axis=1, keepdims=True)
    ex = jnp.exp(logits - lmax)
    exmax = jnp.max(ex, axis=1, keepdims=True)
    gate = exmax / jnp.sum(ex, axis=1, keepdims=True)
    lane = lax.broadcasted_iota(jnp.int32, (_TB, _E), 1).astype(jnp.float32)
    idx_f = jnp.min(jnp.where(ex == exmax, lane, float(_E)),
                    axis=1, keepdims=True)
    oh = (lane == idx_f).astype(jnp.float32)

    r = lax.broadcasted_iota(jnp.int32, (_TB, _TB), 0)
    c = lax.broadcasted_iota(jnp.int32, (_TB, _TB), 1)
    ltri = (c < r).astype(jnp.float32)
    prefix = jnp.dot(ltri, oh, preferred_element_type=jnp.float32)

    cnt = cnt_scr[...]
    pos = jnp.sum(oh * (prefix + cnt), axis=1, keepdims=True)
    slot_f = jnp.where(pos < float(_C), idx_f * float(_C) + pos,
                       float(_TRASH))
    slot_ref[pl.ds(i * (_TB // 128), _TB // 128), :] = (
        slot_f.astype(jnp.int32).reshape(_TB // 128, 128))

    b_idx = (i * _TB >= _S).astype(jnp.int32)
    col = lax.broadcasted_iota(jnp.int32, (_TB, _WCOLS), 1)
    wbrow_ref[...] = jnp.where(col == b_idx, gate, 0.0)

    new_cnt = cnt + jnp.sum(oh, axis=0, keepdims=True)
    cnt_scr[...] = new_cnt
    counts_ref[...] = new_cnt.astype(jnp.int32)


_router = pl.pallas_call(
    _router_body,
    grid=(_NBLK,),
    in_specs=[
        pl.BlockSpec((_TB, _D), lambda i: (i, 0)),
        pl.BlockSpec((_D, _E), lambda i: (0, 0)),
    ],
    out_specs=[
        pl.BlockSpec((_T // 128, 128), lambda i: (0, 0)),
        pl.BlockSpec((_TB, _WCOLS), lambda i: (i, 0)),
        pl.BlockSpec((1, _E), lambda i: (0, 0)),
    ],
    out_shape=[
        jax.ShapeDtypeStruct((_T // 128, 128), jnp.int32),
        jax.ShapeDtypeStruct((_T, _WCOLS), jnp.float32),
        jax.ShapeDtypeStruct((1, _E), jnp.int32),
    ],
    scratch_shapes=[pltpu.VMEM((1, _E), jnp.float32)],
)


_HT = _TPW // 2


def _dispatch_body(t_hbm, wbrow_hbm, slot_hbm, disp_hbm, wb_hbm,
                   idx_v, rows_v, wbr_v, sem1, sem2, sem3, sem4):
    wid = lax.axis_index("s") * 2 + lax.axis_index("c")
    base = wid * _TPW
    ld1a = pltpu.async_copy(slot_hbm.at[pl.ds(base, _HT)], idx_v.at[0], sem1)
    ld1b = pltpu.async_copy(slot_hbm.at[pl.ds(base + _HT, _HT)],
                            idx_v.at[1], sem1)
    ldr0 = pltpu.async_copy(t_hbm.at[pl.ds(base, _HT)], rows_v.at[0], sem2)
    ldw0 = pltpu.async_copy(wbrow_hbm.at[pl.ds(base, _HT)], wbr_v.at[0], sem3)
    ldw1 = pltpu.async_copy(wbrow_hbm.at[pl.ds(base + _HT, _HT)],
                            wbr_v.at[1], sem3)
    ldr1 = pltpu.async_copy(t_hbm.at[pl.ds(base + _HT, _HT)], rows_v.at[1],
                            sem4)
    ld1a.wait()
    ld1b.wait()
    ldr0.wait()
    cp0 = pltpu.async_copy(rows_v.at[0], disp_hbm.at[idx_v.at[0]], sem2)
    ldw0.wait()
    ldw1.wait()
    cpw0 = pltpu.async_copy(wbr_v.at[0], wb_hbm.at[idx_v.at[0]], sem3)
    cpw1 = pltpu.async_copy(wbr_v.at[1], wb_hbm.at[idx_v.at[1]], sem3)
    ldr1.wait()
    cp1 = pltpu.async_copy(rows_v.at[1], disp_hbm.at[idx_v.at[1]], sem4)
    cp0.wait()
    cpw0.wait()
    cpw1.wait()
    cp1.wait()


@functools.cache
def _make_dispatch():
    return pl.kernel(
        _dispatch_body,
        out_type=[
            jax.ShapeDtypeStruct((_TRASH + 1, _D), jnp.float32),
            jax.ShapeDtypeStruct((_TRASH + 1, _WCOLS), jnp.float32),
        ],
        mesh=plsc.VectorSubcoreMesh(core_axis_name="c",
                                    subcore_axis_name="s"),
        scratch_types=[
            pltpu.VMEM((2, _HT), jnp.int32),
            pltpu.VMEM((2, _HT, _D), jnp.float32),
            pltpu.VMEM((2, _HT, _WCOLS), jnp.float32),
            pltpu.SemaphoreType.DMA,
            pltpu.SemaphoreType.DMA,
            pltpu.SemaphoreType.DMA,
            pltpu.SemaphoreType.DMA,
        ],
    )


_EB = 2


def _ffn_body(cnt_smem, disp_ref, wb_ref, w1_ref, b1_ref, w2_ref, b2_ref,
              wc_ref, bc_ref, out_ref, acc):
    i = pl.program_id(0)
    contrib = None
    for j in range(_EB):
        nfill = jnp.minimum(cnt_smem[i * _EB + j], _C)
        rmask_d = lax.broadcasted_iota(jnp.int32, (_C, _D), 0) < nfill
        dispc = jnp.where(rmask_d, disp_ref[pl.ds(j * _C, _C), :], 0.0)
        rmask_w = lax.broadcasted_iota(jnp.int32, (_C, _WCOLS), 0) < nfill
        wbc = jnp.where(rmask_w, wb_ref[pl.ds(j * _C, _C), :], 0.0)

        h = jnp.maximum(
            jnp.dot(dispc, w1_ref[j], preferred_element_type=jnp.float32)
            + b1_ref[j], 0.0)
        hb = lax.dot_general(wbc, h, (((0,), (0,)), ((), ())),
                             preferred_element_type=jnp.float32)
        cj = jnp.dot(hb, w2_ref[j], preferred_element_type=jnp.float32)
        ones_col = jnp.ones((_C, 1), jnp.float32)
        s_col = lax.dot_general(wbc, ones_col, (((0,), (0,)), ((), ())),
                                preferred_element_type=jnp.float32)
        cj = cj + s_col * b2_ref[j]
        contrib = cj if contrib is None else contrib + cj

    @pl.when(i == 0)
    def _():
        acc[...] = contrib

    @pl.when(i != 0)
    def _():
        acc[...] = acc[...] + contrib

    @pl.when(i == _E // _EB - 1)
    def _():
        pooled = acc[...] * (1.0 / _S)
        logits = (jnp.dot(pooled, wc_ref[...],
                          preferred_element_type=jnp.float32)
                  + bc_ref[...])
        out_ref[...] = logits[0:_B, :]


_ffn = pl.pallas_call(
    _ffn_body,
    grid_spec=pltpu.PrefetchScalarGridSpec(
        num_scalar_prefetch=1,
        grid=(_E // _EB,),
        in_specs=[
            pl.BlockSpec((_EB * _C, _D), lambda e, cnt: (e, 0)),
            pl.BlockSpec((_EB * _C, _WCOLS), lambda e, cnt: (e, 0)),
            pl.BlockSpec((_EB, _D, _DFF), lambda e, cnt: (e, 0, 0)),
            pl.BlockSpec((_EB, 1, _DFF), lambda e, cnt: (e, 0, 0)),
            pl.BlockSpec((_EB, _DFF, _D), lambda e, cnt: (e, 0, 0)),
            pl.BlockSpec((_EB, 1, _D), lambda e, cnt: (e, 0, 0)),
            pl.BlockSpec((_D, _NC), lambda e, cnt: (0, 0)),
            pl.BlockSpec((1, _NC), lambda e, cnt: (0, 0)),
        ],
        out_specs=pl.BlockSpec((_B, _NC), lambda e, cnt: (0, 0)),
        scratch_shapes=[pltpu.VMEM((_WCOLS, _D), jnp.float32)],
    ),
    out_shape=jax.ShapeDtypeStruct((_B, _NC), jnp.float32),
)


def kernel(x, Wg, W1, b1, W2, b2, Wc, bc):
    t = x.reshape(_T, _D)
    slot2, wbrow, counts = _router(t, Wg)
    slot = slot2.reshape(_T)
    cnt_i = counts.reshape(_E)
    disp, wb = _make_dispatch()(t, wbrow, slot)
    return _ffn(cnt_i, disp, wb, W1, b1.reshape(_E, 1, _DFF), W2,
                b2.reshape(_E, 1, _D), Wc, bc.reshape(1, _NC))
